# baseline (speedup 1.0000x reference)
import functools

import jax
import jax.numpy as jnp
from jax import lax
from jax.experimental import pallas as pl
from jax.experimental.pallas import tpu as pltpu
from jax.experimental.pallas import tpu_sc as plsc

_N = 8192
_NC = 2
_NS = 16
_NW = _NC * _NS
_ROWS = _N // _NW
_L = 16
_IB = 4
_NJ = _N // _L


def _bcast(vec, idx):
    idxv = jnp.full((_L, 1), idx, jnp.int32)
    return lax.gather(
        vec,
        idxv,
        lax.GatherDimensionNumbers(
            offset_dims=(), collapsed_slice_dims=(0,), start_index_map=(0,)
        ),
        (1,),
        mode=lax.GatherScatterMode.PROMISE_IN_BOUNDS,
    )


def _sc_body(pa_hbm, pb_hbm, rmin_hbm, cpart_hbm, pa_v, b_v, cmin_v, rm_v):
    wid = lax.axis_index("s") * _NC + lax.axis_index("c")
    base = wid * _ROWS

    for r in range(4):
        pltpu.sync_copy(pa_hbm.at[pl.ds(r * _N + base, _ROWS)], pa_v.at[r])
        pltpu.sync_copy(pb_hbm.at[pl.ds(r * _N, _N)], b_v.at[r])

    inf16 = jnp.full((_L,), jnp.inf, jnp.float32)

    def init_cmin(jc, _):
        cmin_v[0, pl.ds(jc * _L, _L)] = inf16
        return 0

    lax.fori_loop(0, _NJ, init_cmin, 0)

    def g_step(g, _):
        i0 = g * _IB
        blk = i0 // _L
        off = i0 % _L
        vecs = [pa_v[r, pl.ds(blk * _L, _L)] for r in range(4)]
        brd = []
        for u in range(_IB):
            brd.append(tuple(_bcast(vecs[r], off + u) for r in range(4)))

        def j_step(jc, rmins):
            jsl = pl.ds(jc * _L, _L)
            vbx = b_v[0, jsl]
            vby = b_v[1, jsl]
            vbz = b_v[2, jsl]
            vbq = b_v[3, jsl]
            cold = cmin_v[0, jsl]
            new_rmins = []
            for u in range(_IB):
                px, py, pz, pq = brd[u]
                h = px * vbx + py * vby + pz * vbz + vbq
                new_rmins.append(jnp.minimum(rmins[u], h))
                cold = jnp.minimum(cold, h + pq)
            cmin_v[0, jsl] = cold
            return tuple(new_rmins)

        rmins = lax.fori_loop(0, _NJ, j_step, (inf16,) * _IB)

        for u in range(_IB):
            rm_v[0, pl.ds((i0 + u) * _L, _L)] = rmins[u]
        return 0

    lax.fori_loop(0, _ROWS // _IB, g_step, 0)

    pltpu.sync_copy(rm_v.at[0], rmin_hbm.at[pl.ds(base * _L, _ROWS * _L)])
    pltpu.sync_copy(cmin_v.at[0], cpart_hbm.at[pl.ds(wid * _N, _N)])


def _combine_body(rminv, asq, cpart, out_ref):
    rmin = jnp.min(rminv[...], axis=1, keepdims=True)
    rd2 = jnp.maximum(rmin + asq[...], 0.0)
    row_sum = jnp.sum(jnp.sqrt(rd2))
    cmin = jnp.min(cpart[...], axis=0, keepdims=True)
    col_sum = jnp.sum(jnp.sqrt(jnp.maximum(cmin, 0.0)))
    out_ref[0, 0] = (row_sum + col_sum) / jnp.float32(_N)


def _rn_bf16(x):
    u = lax.bitcast_convert_type(x, jnp.uint32)
    u = (u + jnp.uint32(0x7FFF) + ((u >> 16) & jnp.uint32(1))) & jnp.uint32(
        0xFFFF0000
    )
    return lax.bitcast_convert_type(u, jnp.float32)


@jax.jit
def kernel(pc1, pc2):
    a = pc1.reshape(-1, 3)
    b = pc2.reshape(-1, 3)
    asq = jnp.sum(a * a, axis=1)
    bsq = jnp.sum(b * b, axis=1)
    a16 = _rn_bf16(a) * jnp.float32(-2.0)
    b16 = _rn_bf16(b)
    pa = jnp.concatenate([a16.T.reshape(-1), asq])
    pb = jnp.concatenate([b16.T.reshape(-1), bsq])

    mesh = plsc.VectorSubcoreMesh(core_axis_name="c", subcore_axis_name="s")
    sc = functools.partial(
        pl.kernel,
        mesh=mesh,
        out_type=[
            jax.ShapeDtypeStruct((_N * _L,), jnp.float32),
            jax.ShapeDtypeStruct((_NW * _N,), jnp.float32),
        ],
        scratch_types=[
            pltpu.VMEM((4, _ROWS), jnp.float32),
            pltpu.VMEM((4, _N), jnp.float32),
            pltpu.VMEM((1, _N), jnp.float32),
            pltpu.VMEM((1, _ROWS * _L), jnp.float32),
        ],
    )(_sc_body)
    rminv, cpart = sc(pa, pb)

    out = pl.pallas_call(
        _combine_body,
        out_shape=jax.ShapeDtypeStruct((1, 1), jnp.float32),
        in_specs=[pl.BlockSpec(memory_space=pltpu.VMEM)] * 3,
        out_specs=pl.BlockSpec(memory_space=pltpu.SMEM),
    )(rminv.reshape(_N, _L), asq.reshape(_N, 1), cpart.reshape(_NW, _N))
    return out[0, 0]


# speedup vs baseline: 4.6690x; 4.6690x over previous
import functools

import jax
import jax.numpy as jnp
from jax import lax
from jax.experimental import pallas as pl
from jax.experimental.pallas import tpu as pltpu
from jax.experimental.pallas import tpu_sc as plsc

_N = 8192
_R0 = 6656
_TAIL = _N - _R0
_BI = 512

_NC = 2
_NS = 16
_NW = _NC * _NS
_ROWS = _TAIL // _NW
_L = 16
_IB = 8
_NJ = _N // _L


def _tc_body(a_ext, b_ext, rsum_ref, cmin_ref):
    ni = _R0 // _BI

    cmin_ref[...] = jnp.full((1, _N), jnp.inf, jnp.float32)

    def i_step(i, row_sum):
        f = jax.lax.dot_general(
            a_ext[pl.ds(i * _BI, _BI), :],
            b_ext[...],
            (((1,), (0,)), ((), ())),
            preferred_element_type=jnp.float32,
        )
        cmin_ref[...] = jnp.minimum(
            cmin_ref[...], jnp.min(f, axis=0, keepdims=True)
        )
        rmin = jnp.maximum(jnp.min(f, axis=1, keepdims=True), 0.0)
        return row_sum + jnp.sum(jnp.sqrt(rmin))

    rsum_ref[0, 0] = jax.lax.fori_loop(0, ni, i_step, jnp.float32(0.0))


def _bcast(vec, idx):
    idxv = jnp.full((_L, 1), idx, jnp.int32)
    return lax.gather(
        vec,
        idxv,
        lax.GatherDimensionNumbers(
            offset_dims=(), collapsed_slice_dims=(0,), start_index_map=(0,)
        ),
        (1,),
        mode=lax.GatherScatterMode.PROMISE_IN_BOUNDS,
    )


def _sc_body(pa_hbm, pb_hbm, rmin_hbm, cpart_hbm, pa_v, b_v, cmin_v, rm_v):
    wid = lax.axis_index("s") * _NC + lax.axis_index("c")
    base = wid * _ROWS

    for r in range(4):
        pltpu.sync_copy(pa_hbm.at[pl.ds(r * _TAIL + base, _ROWS)], pa_v.at[r])
        pltpu.sync_copy(pb_hbm.at[pl.ds(r * _N, _N)], b_v.at[r])

    inf16 = jnp.full((_L,), jnp.inf, jnp.float32)

    def init_cmin(jc, _):
        cmin_v[0, pl.ds(jc * _L, _L)] = inf16
        return 0

    lax.fori_loop(0, _NJ, init_cmin, 0)

    def g_step(g, _):
        i0 = g * _IB
        blk = i0 // _L
        off = i0 % _L
        vecs = [pa_v[r, pl.ds(blk * _L, _L)] for r in range(4)]
        brd = []
        for u in range(_IB):
            brd.append(tuple(_bcast(vecs[r], off + u) for r in range(4)))

        def j_step(jc, rmins):
            jsl = pl.ds(jc * _L, _L)
            vbx = b_v[0, jsl]
            vby = b_v[1, jsl]
            vbz = b_v[2, jsl]
            vbq = b_v[3, jsl]
            cold = cmin_v[0, jsl]
            new_rmins = []
            for u in range(_IB):
                px, py, pz, pq = brd[u]
                h = px * vbx + py * vby + pz * vbz + vbq
                new_rmins.append(jnp.minimum(rmins[u], h))
                cold = jnp.minimum(cold, h + pq)
            cmin_v[0, jsl] = cold
            return tuple(new_rmins)

        rmins = lax.fori_loop(0, _NJ, j_step, (inf16,) * _IB, unroll=2)

        for u in range(_IB):
            rm_v[0, pl.ds((i0 + u) * _L, _L)] = rmins[u]
        return 0

    lax.fori_loop(0, _ROWS // _IB, g_step, 0)

    pltpu.sync_copy(rm_v.at[0], rmin_hbm.at[pl.ds(base * _L, _ROWS * _L)])
    pltpu.sync_copy(cmin_v.at[0], cpart_hbm.at[pl.ds(wid * _N, _N)])


def _combine_body(rsum_tc, cmin_tc, rminv, asq_tail, cpart, out_ref):
    rmin = jnp.min(rminv[...], axis=1, keepdims=True)
    rd2 = jnp.maximum(rmin + asq_tail[...], 0.0)
    row_sum = rsum_tc[0, 0] + jnp.sum(jnp.sqrt(rd2))
    cmin = jnp.minimum(
        cmin_tc[...], jnp.min(cpart[...], axis=0, keepdims=True)
    )
    col_sum = jnp.sum(jnp.sqrt(jnp.maximum(cmin, 0.0)))
    out_ref[0, 0] = (row_sum + col_sum) / jnp.float32(_N)


def _rn_bf16(x):
    u = lax.bitcast_convert_type(x, jnp.uint32)
    u = (u + jnp.uint32(0x7FFF) + ((u >> 16) & jnp.uint32(1))) & jnp.uint32(
        0xFFFF0000
    )
    return lax.bitcast_convert_type(u, jnp.float32)


@jax.jit
def kernel(pc1, pc2):
    a = pc1.reshape(-1, 3)
    b = pc2.reshape(-1, 3)
    asq = jnp.sum(a * a, axis=1, keepdims=True)
    bsq = jnp.sum(b * b, axis=1, keepdims=True)
    asq_hi = _rn_bf16(asq)
    asq_lo = asq - asq_hi
    bsq_hi = _rn_bf16(bsq)
    bsq_lo = bsq - bsq_hi
    a16 = _rn_bf16(a) * jnp.float32(-2.0)
    b16 = _rn_bf16(b)
    ones = jnp.ones_like(asq)
    zeros = jnp.zeros_like(asq)
    a_ext = jnp.concatenate(
        [a16, asq_hi, asq_lo, ones, ones, zeros], axis=1
    )[:_R0].astype(jnp.bfloat16)
    b_ext = (
        jnp.concatenate([b16, ones, ones, bsq_hi, bsq_lo, zeros], axis=1)
        .astype(jnp.bfloat16)
        .T
    )
    pa = jnp.concatenate(
        [a16[_R0:].T.reshape(-1), asq[_R0:, 0]]
    )
    pb = jnp.concatenate([b16.T.reshape(-1), bsq[:, 0]])

    mesh = plsc.VectorSubcoreMesh(core_axis_name="c", subcore_axis_name="s")
    sc = functools.partial(
        pl.kernel,
        mesh=mesh,
        out_type=[
            jax.ShapeDtypeStruct((_TAIL * _L,), jnp.float32),
            jax.ShapeDtypeStruct((_NW * _N,), jnp.float32),
        ],
        scratch_types=[
            pltpu.VMEM((4, _ROWS), jnp.float32),
            pltpu.VMEM((4, _N), jnp.float32),
            pltpu.VMEM((1, _N), jnp.float32),
            pltpu.VMEM((1, _ROWS * _L), jnp.float32),
        ],
    )(_sc_body)
    rminv, cpart = sc(pa, pb)

    rsum_tc, cmin_tc = pl.pallas_call(
        _tc_body,
        out_shape=[
            jax.ShapeDtypeStruct((1, 1), jnp.float32),
            jax.ShapeDtypeStruct((1, _N), jnp.float32),
        ],
        in_specs=[pl.BlockSpec(memory_space=pltpu.VMEM)] * 2,
        out_specs=[
            pl.BlockSpec(memory_space=pltpu.SMEM),
            pl.BlockSpec(memory_space=pltpu.VMEM),
        ],
    )(a_ext, b_ext)

    out = pl.pallas_call(
        _combine_body,
        out_shape=jax.ShapeDtypeStruct((1, 1), jnp.float32),
        in_specs=[
            pl.BlockSpec(memory_space=pltpu.SMEM),
            pl.BlockSpec(memory_space=pltpu.VMEM),
            pl.BlockSpec(memory_space=pltpu.VMEM),
            pl.BlockSpec(memory_space=pltpu.VMEM),
            pl.BlockSpec(memory_space=pltpu.VMEM),
        ],
        out_specs=pl.BlockSpec(memory_space=pltpu.SMEM),
    )(
        rsum_tc,
        cmin_tc,
        rminv.reshape(_TAIL, _L),
        asq[_R0:],
        cpart.reshape(_NW, _N),
    )
    return out[0, 0]


# speedup vs baseline: 6.5029x; 1.3928x over previous
import functools

import jax
import jax.numpy as jnp
from jax import lax
from jax.experimental import pallas as pl
from jax.experimental.pallas import tpu as pltpu
from jax.experimental.pallas import tpu_sc as plsc

_N = 8192
_R0 = 7680
_TAIL = _N - _R0
_BI = 512

_NC = 2
_NS = 16
_NW = _NC * _NS
_ROWS = _TAIL // _NW
_L = 16
_IB = 8
_NJ = _N // _L


def _tc_body(a_ext, b_ext, rsum_ref, cmin_ref):
    ni = _R0 // _BI

    cmin_ref[...] = jnp.full((1, _N), jnp.inf, jnp.float32)

    def i_step(i, row_sum):
        f = jax.lax.dot_general(
            a_ext[pl.ds(i * _BI, _BI), :],
            b_ext[...],
            (((1,), (0,)), ((), ())),
            preferred_element_type=jnp.float32,
        )
        cmin_ref[...] = jnp.minimum(
            cmin_ref[...], jnp.min(f, axis=0, keepdims=True)
        )
        rmin = jnp.maximum(jnp.min(f, axis=1, keepdims=True), 0.0)
        return row_sum + jnp.sum(jnp.sqrt(rmin))

    rsum_ref[0, 0] = jax.lax.fori_loop(0, ni, i_step, jnp.float32(0.0))


def _bcast(vec, idx):
    idxv = jnp.full((_L, 1), idx, jnp.int32)
    return lax.gather(
        vec,
        idxv,
        lax.GatherDimensionNumbers(
            offset_dims=(), collapsed_slice_dims=(0,), start_index_map=(0,)
        ),
        (1,),
        mode=lax.GatherScatterMode.PROMISE_IN_BOUNDS,
    )


def _sc_body(pa_hbm, pb_hbm, rmin_hbm, cpart_hbm, pa_v, b_v, cmin_v, rm_v):
    wid = lax.axis_index("s") * _NC + lax.axis_index("c")
    base = wid * _ROWS

    for r in range(4):
        pltpu.sync_copy(pa_hbm.at[pl.ds(r * _TAIL + base, _ROWS)], pa_v.at[r])
        pltpu.sync_copy(pb_hbm.at[pl.ds(r * _N, _N)], b_v.at[r])

    inf16 = jnp.full((_L,), jnp.inf, jnp.float32)

    def init_cmin(jc, _):
        cmin_v[0, pl.ds(jc * _L, _L)] = inf16
        return 0

    lax.fori_loop(0, _NJ, init_cmin, 0)

    def g_step(g, _):
        i0 = g * _IB
        blk = i0 // _L
        off = i0 % _L
        vecs = [pa_v[r, pl.ds(blk * _L, _L)] for r in range(4)]
        brd = []
        for u in range(_IB):
            brd.append(tuple(_bcast(vecs[r], off + u) for r in range(4)))

        def j_step(jc, rmins):
            jsl = pl.ds(jc * _L, _L)
            vbx = b_v[0, jsl]
            vby = b_v[1, jsl]
            vbz = b_v[2, jsl]
            vbq = b_v[3, jsl]
            cold = cmin_v[0, jsl]
            new_rmins = []
            for u in range(_IB):
                px, py, pz, pq = brd[u]
                h = px * vbx + py * vby + pz * vbz + vbq
                new_rmins.append(jnp.minimum(rmins[u], h))
                cold = jnp.minimum(cold, h + pq)
            cmin_v[0, jsl] = cold
            return tuple(new_rmins)

        rmins = lax.fori_loop(0, _NJ, j_step, (inf16,) * _IB, unroll=2)

        for u in range(_IB):
            rm_v[0, pl.ds((i0 + u) * _L, _L)] = rmins[u]
        return 0

    lax.fori_loop(0, _ROWS // _IB, g_step, 0)

    pltpu.sync_copy(rm_v.at[0], rmin_hbm.at[pl.ds(base * _L, _ROWS * _L)])
    pltpu.sync_copy(cmin_v.at[0], cpart_hbm.at[pl.ds(wid * _N, _N)])


def _combine_body(rsum_tc, cmin_tc, rminv, asq_tail, cpart, out_ref):
    rmin = jnp.min(rminv[...], axis=1, keepdims=True)
    rd2 = jnp.maximum(rmin + asq_tail[...], 0.0)
    row_sum = rsum_tc[0, 0] + jnp.sum(jnp.sqrt(rd2))
    cmin = jnp.minimum(
        cmin_tc[...], jnp.min(cpart[...], axis=0, keepdims=True)
    )
    col_sum = jnp.sum(jnp.sqrt(jnp.maximum(cmin, 0.0)))
    out_ref[0, 0] = (row_sum + col_sum) / jnp.float32(_N)


def _rn_bf16(x):
    u = lax.bitcast_convert_type(x, jnp.uint32)
    u = (u + jnp.uint32(0x7FFF) + ((u >> 16) & jnp.uint32(1))) & jnp.uint32(
        0xFFFF0000
    )
    return lax.bitcast_convert_type(u, jnp.float32)


@jax.jit
def kernel(pc1, pc2):
    a = pc1.reshape(-1, 3)
    b = pc2.reshape(-1, 3)
    asq = jnp.sum(a * a, axis=1, keepdims=True)
    bsq = jnp.sum(b * b, axis=1, keepdims=True)
    asq_hi = _rn_bf16(asq)
    asq_lo = asq - asq_hi
    bsq_hi = _rn_bf16(bsq)
    bsq_lo = bsq - bsq_hi
    a16 = _rn_bf16(a) * jnp.float32(-2.0)
    b16 = _rn_bf16(b)
    ones = jnp.ones_like(asq)
    zeros = jnp.zeros_like(asq)
    a_ext = jnp.concatenate(
        [a16, asq_hi, asq_lo, ones, ones, zeros], axis=1
    )[:_R0].astype(jnp.bfloat16)
    b_ext = (
        jnp.concatenate([b16, ones, ones, bsq_hi, bsq_lo, zeros], axis=1)
        .astype(jnp.bfloat16)
        .T
    )
    pa = jnp.concatenate(
        [a16[_R0:].T.reshape(-1), asq[_R0:, 0]]
    )
    pb = jnp.concatenate([b16.T.reshape(-1), bsq[:, 0]])

    mesh = plsc.VectorSubcoreMesh(core_axis_name="c", subcore_axis_name="s")
    sc = functools.partial(
        pl.kernel,
        mesh=mesh,
        out_type=[
            jax.ShapeDtypeStruct((_TAIL * _L,), jnp.float32),
            jax.ShapeDtypeStruct((_NW * _N,), jnp.float32),
        ],
        scratch_types=[
            pltpu.VMEM((4, _ROWS), jnp.float32),
            pltpu.VMEM((4, _N), jnp.float32),
            pltpu.VMEM((1, _N), jnp.float32),
            pltpu.VMEM((1, _ROWS * _L), jnp.float32),
        ],
    )(_sc_body)
    rminv, cpart = sc(pa, pb)

    rsum_tc, cmin_tc = pl.pallas_call(
        _tc_body,
        out_shape=[
            jax.ShapeDtypeStruct((1, 1), jnp.float32),
            jax.ShapeDtypeStruct((1, _N), jnp.float32),
        ],
        in_specs=[pl.BlockSpec(memory_space=pltpu.VMEM)] * 2,
        out_specs=[
            pl.BlockSpec(memory_space=pltpu.SMEM),
            pl.BlockSpec(memory_space=pltpu.VMEM),
        ],
    )(a_ext, b_ext)

    out = pl.pallas_call(
        _combine_body,
        out_shape=jax.ShapeDtypeStruct((1, 1), jnp.float32),
        in_specs=[
            pl.BlockSpec(memory_space=pltpu.SMEM),
            pl.BlockSpec(memory_space=pltpu.VMEM),
            pl.BlockSpec(memory_space=pltpu.VMEM),
            pl.BlockSpec(memory_space=pltpu.VMEM),
            pl.BlockSpec(memory_space=pltpu.VMEM),
        ],
        out_specs=pl.BlockSpec(memory_space=pltpu.SMEM),
    )(
        rsum_tc,
        cmin_tc,
        rminv.reshape(_TAIL, _L),
        asq[_R0:],
        cpart.reshape(_NW, _N),
    )
    return out[0, 0]


# speedup vs baseline: 9.9253x; 1.5263x over previous
import functools

import jax
import jax.numpy as jnp
from jax.experimental import pallas as pl
from jax.experimental.pallas import tpu as pltpu

_N = 8192
_BI = 1024


def _chamfer_body(a_ext, b_ext, out_ref, cmin_ref):
    ni = _N // _BI

    cmin_ref[...] = jnp.full((1, _N), jnp.inf, jnp.float32)

    def i_step(i, row_sum):
        f = jax.lax.dot_general(
            a_ext[pl.ds(i * _BI, _BI), :],
            b_ext[...],
            (((1,), (0,)), ((), ())),
            preferred_element_type=jnp.float32,
        )
        cmin_ref[...] = jnp.minimum(
            cmin_ref[...], jnp.min(f, axis=0, keepdims=True)
        )
        rmin = jnp.maximum(jnp.min(f, axis=1, keepdims=True), 0.0)
        return row_sum + jnp.sum(jnp.sqrt(rmin))

    row_sum = jax.lax.fori_loop(0, ni, i_step, jnp.float32(0.0))
    col_sum = jnp.sum(jnp.sqrt(jnp.maximum(cmin_ref[...], 0.0)))
    out_ref[0, 0] = (row_sum + col_sum) / jnp.float32(_N)


def _rn_bf16(x):
    u = jax.lax.bitcast_convert_type(x, jnp.uint32)
    u = (u + jnp.uint32(0x7FFF) + ((u >> 16) & jnp.uint32(1))) & jnp.uint32(
        0xFFFF0000
    )
    return jax.lax.bitcast_convert_type(u, jnp.float32)


@jax.jit
def kernel(pc1, pc2):
    a = pc1.reshape(-1, 3)
    b = pc2.reshape(-1, 3)
    asq = jnp.sum(a * a, axis=1, keepdims=True)
    bsq = jnp.sum(b * b, axis=1, keepdims=True)
    asq_hi = _rn_bf16(asq)
    asq_lo = asq - asq_hi
    bsq_hi = _rn_bf16(bsq)
    bsq_lo = bsq - bsq_hi
    a16 = _rn_bf16(a) * jnp.float32(-2.0)
    b16 = _rn_bf16(b)
    ones = jnp.ones_like(asq)
    zeros = jnp.zeros_like(asq)
    a_ext = jnp.concatenate(
        [a16, asq_hi, asq_lo, ones, ones, zeros], axis=1
    ).astype(jnp.bfloat16)
    b_ext = (
        jnp.concatenate([b16, ones, ones, bsq_hi, bsq_lo, zeros], axis=1)
        .astype(jnp.bfloat16)
        .T
    )
    out = pl.pallas_call(
        _chamfer_body,
        out_shape=jax.ShapeDtypeStruct((1, 1), jnp.float32),
        in_specs=[pl.BlockSpec(memory_space=pltpu.VMEM)] * 2,
        out_specs=pl.BlockSpec(memory_space=pltpu.SMEM),
        scratch_shapes=[pltpu.VMEM((1, _N), jnp.float32)],
    )(a_ext, b_ext)
    return out[0, 0]


# speedup vs baseline: 10.2218x; 1.0299x over previous
import functools

import jax
import jax.numpy as jnp
from jax.experimental import pallas as pl
from jax.experimental.pallas import tpu as pltpu

_N = 8192
_BI = 2048


def _chamfer_body(a_ext, b_ext, out_ref, cmin_ref):
    ni = _N // _BI

    cmin_ref[...] = jnp.full((1, _N), jnp.inf, jnp.float32)

    def i_step(i, row_sum):
        f = jax.lax.dot_general(
            a_ext[pl.ds(i * _BI, _BI), :],
            b_ext[...],
            (((1,), (0,)), ((), ())),
            preferred_element_type=jnp.float32,
        )
        cmin_ref[...] = jnp.minimum(
            cmin_ref[...], jnp.min(f, axis=0, keepdims=True)
        )
        rmin = jnp.maximum(jnp.min(f, axis=1, keepdims=True), 0.0)
        return row_sum + jnp.sum(jnp.sqrt(rmin))

    row_sum = jax.lax.fori_loop(0, ni, i_step, jnp.float32(0.0))
    col_sum = jnp.sum(jnp.sqrt(jnp.maximum(cmin_ref[...], 0.0)))
    out_ref[0, 0] = (row_sum + col_sum) / jnp.float32(_N)


def _rn_bf16(x):
    u = jax.lax.bitcast_convert_type(x, jnp.uint32)
    u = (u + jnp.uint32(0x7FFF) + ((u >> 16) & jnp.uint32(1))) & jnp.uint32(
        0xFFFF0000
    )
    return jax.lax.bitcast_convert_type(u, jnp.float32)


@jax.jit
def kernel(pc1, pc2):
    a = pc1.reshape(-1, 3)
    b = pc2.reshape(-1, 3)
    asq = jnp.sum(a * a, axis=1, keepdims=True)
    bsq = jnp.sum(b * b, axis=1, keepdims=True)
    asq_hi = _rn_bf16(asq)
    asq_lo = asq - asq_hi
    bsq_hi = _rn_bf16(bsq)
    bsq_lo = bsq - bsq_hi
    a16 = _rn_bf16(a) * jnp.float32(-2.0)
    b16 = _rn_bf16(b)
    ones = jnp.ones_like(asq)
    zeros = jnp.zeros_like(asq)
    a_ext = jnp.concatenate(
        [a16, asq_hi, asq_lo, ones, ones, zeros], axis=1
    ).astype(jnp.bfloat16)
    b_ext = (
        jnp.concatenate([b16, ones, ones, bsq_hi, bsq_lo, zeros], axis=1)
        .astype(jnp.bfloat16)
        .T
    )
    out = pl.pallas_call(
        _chamfer_body,
        out_shape=jax.ShapeDtypeStruct((1, 1), jnp.float32),
        in_specs=[pl.BlockSpec(memory_space=pltpu.VMEM)] * 2,
        out_specs=pl.BlockSpec(memory_space=pltpu.SMEM),
        scratch_shapes=[pltpu.VMEM((1, _N), jnp.float32)],
    )(a_ext, b_ext)
    return out[0, 0]
